# baseline (speedup 1.0000x reference)
import functools

import jax
import jax.numpy as jnp
from jax import lax
from jax.experimental import pallas as pl
from jax.experimental.pallas import tpu as pltpu
from jax.experimental.pallas import tpu_sc as plsc

N = 10000
E = 160000
D_IN = 128
HID = 32
E_IN = 16
E_HID = 128
STEPS = 2
ALPHA = 0.1
BETA = 1.0 / STEPS

SC_CORES = 2
SC_SUBCORES = 16
SC_W = SC_CORES * SC_SUBCORES
EPW = E // SC_W
GCH = 128
NCHM = 39
TAIL = EPW - NCHM * GCH
NPAD = 10240
NPT = NPAD // SC_SUBCORES

TILE_E = 3200

HALF_A = 83200
HALF_B = E - HALF_A
EPWA = HALF_A // SC_W
NCHA = EPWA // GCH
TAILA = EPWA - NCHA * GCH
EPWB = HALF_B // SC_W
NCHB = EPWB // GCH
TAILB = EPWB - NCHB * GCH


def _lin0_body(nf, w0t, b0, out):
    out[...] = jnp.maximum(
        jnp.dot(nf[...], w0t[...], preferred_element_type=jnp.float32) + b0[...],
        0.0)


def _edge_body(ef, hp, we1t, be1, we2t, b2, rrep, stile, msgp):
    hpv = hp[...]
    h = jnp.concatenate([hpv[:, HID * q:HID * (q + 1)] for q in range(4)],
                        axis=0)
    t = jnp.maximum(
        jnp.dot(ef[...], we1t[...], preferred_element_type=jnp.float32) + be1[...],
        0.0)
    wm = jnp.dot(t.astype(jnp.bfloat16), we2t[...],
                 preferred_element_type=jnp.float32)
    hr = jnp.dot(h.astype(jnp.bfloat16), rrep[...],
                 preferred_element_type=jnp.float32)
    prod = hr * wm
    t1 = prod[:, 0:128]
    for k in range(1, 8):
        t1 = t1 + prod[:, k * 128:(k + 1) * 128]
    s = t1[:, 0:64] + t1[:, 64:128]
    m2 = jnp.dot(h, b2[...], preferred_element_type=jnp.float32)
    msg = s[:, 0:32] + s[:, 32:64] + m2
    del stile
    q4 = TILE_E // 4
    msgp[...] = jnp.concatenate([msg[q4 * q:q4 * (q + 1), :] for q in range(4)],
                                axis=1)


def _node_body(a0, a1, b0_, b1_, out, h0, w1t, b1, bc, new):
    agg = (a0[...] + a1[...]) + (b0_[...] + b1_[...])
    conv = agg + out[...] + bc[...]
    temp = ALPHA * conv + (1.0 - ALPHA) * h0[...]
    lin = jnp.dot(temp, w1t[...], preferred_element_type=jnp.float32) + b1[...]
    new[...] = jnp.maximum(BETA * lin + (1.0 - BETA) * temp, 0.0)


def _bn_body(x, gamma, beta_bn, wyt, by, wy2t, by2, y1, y2):
    v = x[...]
    mu = jnp.mean(v, axis=0, keepdims=True)
    d = v - mu
    var = jnp.mean(d * d, axis=0, keepdims=True)
    yb = d * (gamma[...] * lax.rsqrt(var + 1e-5)) + beta_bn[...]
    y1[...] = jax.nn.sigmoid(
        jnp.dot(yb, wyt[...], preferred_element_type=jnp.float32) + by[...])
    y2[...] = jax.nn.sigmoid(
        jnp.dot(yb, wy2t[...], preferred_element_type=jnp.float32) + by2[...])


def _lin0(n_feat, w0t, b0):
    return pl.pallas_call(
        _lin0_body,
        out_shape=jax.ShapeDtypeStruct((N, HID), jnp.float32),
    )(n_feat, w0t, b0)


def _edge(e_feat, hp, we1t, be1, we2t, b2, rrep, stile, half, tile_off):
    grid = (half // TILE_E,)
    fixed = lambda i: (0, 0)
    return pl.pallas_call(
        _edge_body,
        grid=grid,
        in_specs=[
            pl.BlockSpec((TILE_E, E_IN), lambda i: (i + tile_off, 0)),
            pl.BlockSpec((TILE_E // 4, 4 * HID), lambda i: (i, 0)),
            pl.BlockSpec((E_IN, E_HID), fixed),
            pl.BlockSpec((1, E_HID), fixed),
            pl.BlockSpec((E_HID, HID * HID), fixed),
            pl.BlockSpec((HID, HID), fixed),
            pl.BlockSpec((HID, HID * HID), fixed),
            pl.BlockSpec((HID * HID, HID), fixed),
        ],
        out_specs=pl.BlockSpec((TILE_E // 4, 4 * HID), lambda i: (i, 0)),
        out_shape=jax.ShapeDtypeStruct((half // 4, 4 * HID), jnp.float32),
        compiler_params=pltpu.CompilerParams(
            dimension_semantics=("arbitrary",)),
    )(e_feat, hp, we1t, be1, we2t, b2, rrep, stile)


def _node(a0, a1, b0_, b1_, out, h0, w1t, b1, bc):
    return pl.pallas_call(
        _node_body,
        out_shape=jax.ShapeDtypeStruct((N, HID), jnp.float32),
    )(a0, a1, b0_, b1_, out, h0, w1t, b1, bc)


def _bn_heads(x, gamma, beta_bn, wyt, by, wy2t, by2):
    return pl.pallas_call(
        _bn_body,
        out_shape=(jax.ShapeDtypeStruct((N, 2), jnp.float32),
                   jax.ShapeDtypeStruct((N, 2), jnp.float32)),
    )(x, gamma, beta_bn, wyt, by, wy2t, by2)


def _gather_body_fn(epw, nchm, tail):
    def body_fn(table_hbm, idxm_hbm, idxt_hbm, out_hbm, idx_v, idxt_v,
                rows_a, rows_b, rowst_v, gsem_a, gsem_b, ssem_a, ssem_b,
                tsem):
        c = lax.axis_index("c")
        s = lax.axis_index("s")
        wid = s * SC_CORES + c
        base = wid * epw
        pltpu.sync_copy(idxm_hbm.at[wid], idx_v)
        pltpu.sync_copy(idxt_hbm.at[wid], idxt_v)

        def issue(j, buf, sem):
            pltpu.async_copy(table_hbm.at[idx_v.at[j]], buf, sem)

        def wait_g(buf, sem):
            pltpu.make_async_copy(table_hbm.at[idx_v.at[0]], buf, sem).wait()

        def store(j, buf, sem):
            pltpu.async_copy(buf, out_hbm.at[pl.ds(base + j * GCH, GCH)], sem)

        def wait_s(buf, sem):
            pltpu.make_async_copy(out_hbm.at[pl.ds(base, GCH)], buf, sem).wait()

        issue(0, rows_a, gsem_a)
        issue(1, rows_b, gsem_b)

        def body(g, carry):
            j0 = 2 * g
            wait_g(rows_a, gsem_a)
            store(j0, rows_a, ssem_a)
            wait_g(rows_b, gsem_b)
            store(j0 + 1, rows_b, ssem_b)

            @pl.when(j0 + 2 < nchm)
            def _():
                wait_s(rows_a, ssem_a)
                issue(j0 + 2, rows_a, gsem_a)

            @pl.when(j0 + 3 < nchm)
            def _():
                wait_s(rows_b, ssem_b)
                issue(j0 + 3, rows_b, gsem_b)

            return carry

        lax.fori_loop(0, nchm // 2, body, 0)
        if nchm % 2:
            wait_g(rows_a, gsem_a)
            store(nchm - 1, rows_a, ssem_a)
        pltpu.async_copy(table_hbm.at[idxt_v], rowst_v, tsem).wait()
        pltpu.sync_copy(rowst_v, out_hbm.at[pl.ds(base + nchm * GCH, tail)])
        wait_s(rows_a, ssem_a)
        wait_s(rows_b, ssem_b)

    return body_fn


def _scatter_body_fn(epw, nchm, tail):
    def body_fn(msg_hbm, idxm_hbm, idxt_hbm, zer_hbm, out_hbm, idx_v,
                idxt_v, rows_a, rows_b, rowst_v, agg_sh, lsem_a, lsem_b,
                tsem):
        c = lax.axis_index("c")
        s = lax.axis_index("s")
        wid = s * SC_CORES + c
        base = wid * epw
        pltpu.sync_copy(zer_hbm, agg_sh.at[pl.ds(s * NPT, NPT)])
        pltpu.sync_copy(idxm_hbm.at[wid], idx_v)
        pltpu.sync_copy(idxt_hbm.at[wid], idxt_v)
        plsc.subcore_barrier()

        def load(j, buf, sem):
            pltpu.async_copy(msg_hbm.at[pl.ds(base + j * GCH, GCH)], buf, sem)

        def wait_l(buf, sem):
            pltpu.make_async_copy(msg_hbm.at[pl.ds(base, GCH)], buf, sem).wait()

        load(0, rows_a, lsem_a)
        load(1, rows_b, lsem_b)

        def body(g, carry):
            j0 = 2 * g
            wait_l(rows_a, lsem_a)
            pltpu.sync_copy(rows_a, agg_sh.at[idx_v.at[j0]], add=True)

            @pl.when(j0 + 2 < nchm)
            def _():
                load(j0 + 2, rows_a, lsem_a)

            wait_l(rows_b, lsem_b)
            pltpu.sync_copy(rows_b, agg_sh.at[idx_v.at[j0 + 1]], add=True)

            @pl.when(j0 + 3 < nchm)
            def _():
                load(j0 + 3, rows_b, lsem_b)

            return carry

        lax.fori_loop(0, nchm // 2, body, 0)
        if nchm % 2:
            wait_l(rows_a, lsem_a)
            pltpu.sync_copy(rows_a, agg_sh.at[idx_v.at[nchm - 1]], add=True)
        pltpu.async_copy(
            msg_hbm.at[pl.ds(base + nchm * GCH, tail)], rowst_v, tsem).wait()
        pltpu.sync_copy(rowst_v, agg_sh.at[idxt_v], add=True)
        plsc.subcore_barrier()
        pltpu.sync_copy(agg_sh.at[pl.ds(s * NPT, NPT)],
                        out_hbm.at[pl.ds(c * NPAD + s * NPT, NPT)])

    return body_fn


@functools.lru_cache(maxsize=None)
def _sc_kernels():
    mesh = plsc.VectorSubcoreMesh(core_axis_name="c", subcore_axis_name="s")
    params = pltpu.CompilerParams(use_tc_tiling_on_sc=False)

    def make_pair(half, epw, nchm, tail):
        gather = pl.kernel(
            _gather_body_fn(epw, nchm, tail),
            out_type=jax.ShapeDtypeStruct((half, HID), jnp.float32),
            mesh=mesh,
            scratch_types=[
                pltpu.VMEM((nchm, GCH), jnp.int32),
                pltpu.VMEM((tail,), jnp.int32),
                pltpu.VMEM((GCH, HID), jnp.float32),
                pltpu.VMEM((GCH, HID), jnp.float32),
                pltpu.VMEM((tail, HID), jnp.float32),
                pltpu.SemaphoreType.DMA,
                pltpu.SemaphoreType.DMA,
                pltpu.SemaphoreType.DMA,
                pltpu.SemaphoreType.DMA,
                pltpu.SemaphoreType.DMA,
            ],
            compiler_params=params,
        )
        scatter = pl.kernel(
            _scatter_body_fn(epw, nchm, tail),
            out_type=jax.ShapeDtypeStruct((SC_CORES * NPAD, HID), jnp.float32),
            mesh=mesh,
            scratch_types=[
                pltpu.VMEM((nchm, GCH), jnp.int32),
                pltpu.VMEM((tail,), jnp.int32),
                pltpu.VMEM((GCH, HID), jnp.float32),
                pltpu.VMEM((GCH, HID), jnp.float32),
                pltpu.VMEM((tail, HID), jnp.float32),
                pltpu.VMEM_SHARED((NPAD, HID), jnp.float32),
                pltpu.SemaphoreType.DMA,
                pltpu.SemaphoreType.DMA,
                pltpu.SemaphoreType.DMA,
            ],
            compiler_params=params,
        )
        return gather, scatter

    return (make_pair(HALF_A, EPWA, NCHA, TAILA),
            make_pair(HALF_B, EPWB, NCHB, TAILB))


def kernel(n_feat, e_feat, edge_index, W0, b0, We1, be1, We2, be2, bc, W1,
           b1, gamma, beta_bn, Wy, by, Wy2, by2):
    def _to_slots(x):
        return x.reshape(E // TILE_E, 4, TILE_E // 4).transpose(0, 2, 1).reshape(E)

    def _split(w, epw, nchm):
        m = w[:, :nchm * GCH].reshape(SC_W, nchm, GCH)
        return m, w[:, nchm * GCH:]

    src_s = _to_slots(edge_index[0])
    dst_s = _to_slots(edge_index[1])
    srcmA, srctA = _split(src_s[:HALF_A].reshape(SC_W, EPWA), EPWA, NCHA)
    srcmB, srctB = _split(src_s[HALF_A:].reshape(SC_W, EPWB), EPWB, NCHB)
    dstmA, dsttA = _split(dst_s[:HALF_A].reshape(SC_W, EPWA), EPWA, NCHA)
    dstmB, dsttB = _split(dst_s[HALF_A:].reshape(SC_W, EPWB), EPWB, NCHB)

    w0t = W0.T
    we1t = We1.T
    we2t = We2.T.astype(jnp.bfloat16)
    w1t = W1.T
    wyt = Wy.T
    wy2t = Wy2.T
    b0r = b0.reshape(1, HID)
    be1r = be1.reshape(1, E_HID)
    b2 = be2.reshape(HID, HID)
    b1r = b1.reshape(1, HID)
    bcr = bc.reshape(1, HID)
    gr = gamma.reshape(1, HID)
    betar = beta_bn.reshape(1, HID)
    byr = by.reshape(1, 2)
    by2r = by2.reshape(1, 2)
    rrep = jnp.repeat(jnp.eye(HID, dtype=jnp.bfloat16), HID, axis=1)
    stile = jnp.tile(jnp.eye(HID, dtype=jnp.bfloat16), (HID, 1))
    zer = jnp.zeros((NPT, HID), jnp.float32)

    (gA, sA), (gB, sB) = _sc_kernels()
    out = _lin0(n_feat, w0t, b0r)
    h0 = out
    for _ in range(STEPS):
        hpA = gA(out, srcmA, srctA).reshape(HALF_A // 4, 4 * HID)
        hpB = gB(out, srcmB, srctB).reshape(HALF_B // 4, 4 * HID)
        mA = _edge(e_feat, hpA, we1t, be1r, we2t, b2, rrep, stile, HALF_A, 0)
        pA = sA(mA.reshape(HALF_A, HID), dstmA, dsttA,
                zer).reshape(SC_CORES, NPAD, HID)
        mB = _edge(e_feat, hpB, we1t, be1r, we2t, b2, rrep, stile, HALF_B,
                   HALF_A // TILE_E)
        pB = sB(mB.reshape(HALF_B, HID), dstmB, dsttB,
                zer).reshape(SC_CORES, NPAD, HID)
        out = _node(pA[0, :N], pA[1, :N], pB[0, :N], pB[1, :N], out, h0,
                    w1t, b1r, bcr)
    return _bn_heads(out, gr, betar, wyt, byr, wy2t, by2r)


# speedup vs baseline: 1.0936x; 1.0936x over previous
import functools

import jax
import jax.numpy as jnp
from jax import lax
from jax.experimental import pallas as pl
from jax.experimental.pallas import tpu as pltpu
from jax.experimental.pallas import tpu_sc as plsc

N = 10000
E = 160000
D_IN = 128
HID = 32
E_IN = 16
E_HID = 128
STEPS = 2
ALPHA = 0.1
BETA = 1.0 / STEPS

SC_CORES = 2
SC_SUBCORES = 16
SC_W = SC_CORES * SC_SUBCORES
EPW = E // SC_W
GCH = 128
NCHM = 39
TAIL = EPW - NCHM * GCH
NPAD = 10240
NPT = NPAD // SC_SUBCORES

TILE_E = 3200

HALF_A = 83200
HALF_B = E - HALF_A
EPWA = HALF_A // SC_W
NCHA = EPWA // GCH
TAILA = EPWA - NCHA * GCH
EPWB = HALF_B // SC_W
NCHB = EPWB // GCH
TAILB = EPWB - NCHB * GCH


def _lin0_body(nf, w0t, b0, out):
    out[...] = jnp.maximum(
        jnp.dot(nf[...], w0t[...], preferred_element_type=jnp.float32) + b0[...],
        0.0)


def _edge_body(ef, hp, we1t, be1, we2t, b2, rrep, stile, msgp):
    hpv = hp[...]
    h = jnp.concatenate([hpv[:, HID * q:HID * (q + 1)] for q in range(4)],
                        axis=0)
    t = jnp.maximum(
        jnp.dot(ef[...], we1t[...], preferred_element_type=jnp.float32) + be1[...],
        0.0)
    wm = jnp.dot(t.astype(jnp.bfloat16), we2t[...],
                 preferred_element_type=jnp.float32)
    hr = jnp.dot(h.astype(jnp.bfloat16), rrep[...],
                 preferred_element_type=jnp.float32)
    prod = hr * wm
    t1 = prod[:, 0:128]
    for k in range(1, 8):
        t1 = t1 + prod[:, k * 128:(k + 1) * 128]
    s = t1[:, 0:64] + t1[:, 64:128]
    m2 = jnp.dot(h, b2[...], preferred_element_type=jnp.float32)
    msg = s[:, 0:32] + s[:, 32:64] + m2
    del stile
    q4 = TILE_E // 4
    msgp[...] = jnp.concatenate([msg[q4 * q:q4 * (q + 1), :] for q in range(4)],
                                axis=1)


def _node_body(a0, a1, b0_, b1_, out, h0, w1t, b1, bc, new):
    agg = (a0[...] + a1[...]) + (b0_[...] + b1_[...])
    conv = agg + out[...] + bc[...]
    temp = ALPHA * conv + (1.0 - ALPHA) * h0[...]
    lin = jnp.dot(temp, w1t[...], preferred_element_type=jnp.float32) + b1[...]
    new[...] = jnp.maximum(BETA * lin + (1.0 - BETA) * temp, 0.0)


def _bn_body(x, gamma, beta_bn, wyt, by, wy2t, by2, y1, y2):
    v = x[...]
    mu = jnp.mean(v, axis=0, keepdims=True)
    d = v - mu
    var = jnp.mean(d * d, axis=0, keepdims=True)
    yb = d * (gamma[...] * lax.rsqrt(var + 1e-5)) + beta_bn[...]
    y1[...] = jax.nn.sigmoid(
        jnp.dot(yb, wyt[...], preferred_element_type=jnp.float32) + by[...])
    y2[...] = jax.nn.sigmoid(
        jnp.dot(yb, wy2t[...], preferred_element_type=jnp.float32) + by2[...])


def _lin0(n_feat, w0t, b0):
    return pl.pallas_call(
        _lin0_body,
        out_shape=jax.ShapeDtypeStruct((N, HID), jnp.float32),
    )(n_feat, w0t, b0)


def _edge(e_feat, hp, we1t, be1, we2t, b2, rrep, stile, half, tile_off):
    grid = (half // TILE_E,)
    fixed = lambda i: (0, 0)
    return pl.pallas_call(
        _edge_body,
        grid=grid,
        in_specs=[
            pl.BlockSpec((TILE_E, E_IN), lambda i: (i + tile_off, 0)),
            pl.BlockSpec((TILE_E // 4, 4 * HID), lambda i: (i, 0)),
            pl.BlockSpec((E_IN, E_HID), fixed),
            pl.BlockSpec((1, E_HID), fixed),
            pl.BlockSpec((E_HID, HID * HID), fixed),
            pl.BlockSpec((HID, HID), fixed),
            pl.BlockSpec((HID, HID * HID), fixed),
            pl.BlockSpec((HID * HID, HID), fixed),
        ],
        out_specs=pl.BlockSpec((TILE_E // 4, 4 * HID), lambda i: (i, 0)),
        out_shape=jax.ShapeDtypeStruct((half // 4, 4 * HID), jnp.float32),
        compiler_params=pltpu.CompilerParams(
            dimension_semantics=("arbitrary",)),
    )(e_feat, hp, we1t, be1, we2t, b2, rrep, stile)


def _node(a0, a1, b0_, b1_, out, h0, w1t, b1, bc):
    return pl.pallas_call(
        _node_body,
        out_shape=jax.ShapeDtypeStruct((N, HID), jnp.float32),
    )(a0, a1, b0_, b1_, out, h0, w1t, b1, bc)


def _bn_heads(x, gamma, beta_bn, wyt, by, wy2t, by2):
    return pl.pallas_call(
        _bn_body,
        out_shape=(jax.ShapeDtypeStruct((N, 2), jnp.float32),
                   jax.ShapeDtypeStruct((N, 2), jnp.float32)),
    )(x, gamma, beta_bn, wyt, by, wy2t, by2)


def _gather_body_fn(epw, nchm, tail):
    def body_fn(table_hbm, idxm_hbm, idxt_hbm, out_hbm, idx_v, idxt_v,
                rows_a, rows_b, rowst_v, gsem_a, gsem_b, ssem_a, ssem_b,
                tsem):
        c = lax.axis_index("c")
        s = lax.axis_index("s")
        wid = s * SC_CORES + c
        base = wid * epw
        pltpu.sync_copy(idxm_hbm.at[wid], idx_v)
        pltpu.sync_copy(idxt_hbm.at[wid], idxt_v)

        def issue(j, buf, sem):
            pltpu.async_copy(table_hbm.at[idx_v.at[j]], buf, sem)

        def wait_g(buf, sem):
            pltpu.make_async_copy(table_hbm.at[idx_v.at[0]], buf, sem).wait()

        def store(j, buf, sem):
            pltpu.async_copy(buf, out_hbm.at[pl.ds(base + j * GCH, GCH)], sem)

        def wait_s(buf, sem):
            pltpu.make_async_copy(out_hbm.at[pl.ds(base, GCH)], buf, sem).wait()

        issue(0, rows_a, gsem_a)
        issue(1, rows_b, gsem_b)

        def body(g, carry):
            j0 = 2 * g
            wait_g(rows_a, gsem_a)
            store(j0, rows_a, ssem_a)
            wait_g(rows_b, gsem_b)
            store(j0 + 1, rows_b, ssem_b)

            @pl.when(j0 + 2 < nchm)
            def _():
                wait_s(rows_a, ssem_a)
                issue(j0 + 2, rows_a, gsem_a)

            @pl.when(j0 + 3 < nchm)
            def _():
                wait_s(rows_b, ssem_b)
                issue(j0 + 3, rows_b, gsem_b)

            return carry

        lax.fori_loop(0, nchm // 2, body, 0)
        if nchm % 2:
            wait_g(rows_a, gsem_a)
            store(nchm - 1, rows_a, ssem_a)
        pltpu.async_copy(table_hbm.at[idxt_v], rowst_v, tsem).wait()
        pltpu.sync_copy(rowst_v, out_hbm.at[pl.ds(base + nchm * GCH, tail)])
        wait_s(rows_a, ssem_a)
        wait_s(rows_b, ssem_b)

    return body_fn


def _scatter_body_fn(epw, nchm, tail):
    def body_fn(msg_hbm, idxm_hbm, idxt_hbm, zer_hbm, out_hbm, idx_v,
                idxt_v, rows_a, rows_b, rowst_v, agg_sh, lsem_a, lsem_b,
                tsem):
        c = lax.axis_index("c")
        s = lax.axis_index("s")
        wid = s * SC_CORES + c
        base = wid * epw
        pltpu.sync_copy(zer_hbm, agg_sh.at[pl.ds(s * NPT, NPT)])
        pltpu.sync_copy(idxm_hbm.at[wid], idx_v)
        pltpu.sync_copy(idxt_hbm.at[wid], idxt_v)
        plsc.subcore_barrier()

        def load(j, buf, sem):
            pltpu.async_copy(msg_hbm.at[pl.ds(base + j * GCH, GCH)], buf, sem)

        def wait_l(buf, sem):
            pltpu.make_async_copy(msg_hbm.at[pl.ds(base, GCH)], buf, sem).wait()

        load(0, rows_a, lsem_a)
        load(1, rows_b, lsem_b)

        def body(g, carry):
            j0 = 2 * g
            wait_l(rows_a, lsem_a)
            pltpu.sync_copy(rows_a, agg_sh.at[idx_v.at[j0]], add=True)

            @pl.when(j0 + 2 < nchm)
            def _():
                load(j0 + 2, rows_a, lsem_a)

            wait_l(rows_b, lsem_b)
            pltpu.sync_copy(rows_b, agg_sh.at[idx_v.at[j0 + 1]], add=True)

            @pl.when(j0 + 3 < nchm)
            def _():
                load(j0 + 3, rows_b, lsem_b)

            return carry

        lax.fori_loop(0, nchm // 2, body, 0)
        if nchm % 2:
            wait_l(rows_a, lsem_a)
            pltpu.sync_copy(rows_a, agg_sh.at[idx_v.at[nchm - 1]], add=True)
        pltpu.async_copy(
            msg_hbm.at[pl.ds(base + nchm * GCH, tail)], rowst_v, tsem).wait()
        pltpu.sync_copy(rowst_v, agg_sh.at[idxt_v], add=True)
        plsc.subcore_barrier()
        pltpu.sync_copy(agg_sh.at[pl.ds(s * NPT, NPT)],
                        out_hbm.at[pl.ds(c * NPAD + s * NPT, NPT)])

    return body_fn


@functools.lru_cache(maxsize=None)
def _sc_kernels():
    mesh = plsc.VectorSubcoreMesh(core_axis_name="c", subcore_axis_name="s")
    params = pltpu.CompilerParams(use_tc_tiling_on_sc=False)

    def make_pair(half, epw, nchm, tail):
        gather = pl.kernel(
            _gather_body_fn(epw, nchm, tail),
            out_type=jax.ShapeDtypeStruct((half, HID), jnp.float32),
            mesh=mesh,
            scratch_types=[
                pltpu.VMEM((nchm, GCH), jnp.int32),
                pltpu.VMEM((tail,), jnp.int32),
                pltpu.VMEM((GCH, HID), jnp.float32),
                pltpu.VMEM((GCH, HID), jnp.float32),
                pltpu.VMEM((tail, HID), jnp.float32),
                pltpu.SemaphoreType.DMA,
                pltpu.SemaphoreType.DMA,
                pltpu.SemaphoreType.DMA,
                pltpu.SemaphoreType.DMA,
                pltpu.SemaphoreType.DMA,
            ],
            compiler_params=params,
        )
        scatter = pl.kernel(
            _scatter_body_fn(epw, nchm, tail),
            out_type=jax.ShapeDtypeStruct((SC_CORES * NPAD, HID), jnp.float32),
            mesh=mesh,
            scratch_types=[
                pltpu.VMEM((nchm, GCH), jnp.int32),
                pltpu.VMEM((tail,), jnp.int32),
                pltpu.VMEM((GCH, HID), jnp.float32),
                pltpu.VMEM((GCH, HID), jnp.float32),
                pltpu.VMEM((tail, HID), jnp.float32),
                pltpu.VMEM_SHARED((NPAD, HID), jnp.float32),
                pltpu.SemaphoreType.DMA,
                pltpu.SemaphoreType.DMA,
                pltpu.SemaphoreType.DMA,
            ],
            compiler_params=params,
        )
        return gather, scatter

    return (make_pair(HALF_A, EPWA, NCHA, TAILA),
            make_pair(HALF_B, EPWB, NCHB, TAILB))


def kernel(n_feat, e_feat, edge_index, W0, b0, We1, be1, We2, be2, bc, W1,
           b1, gamma, beta_bn, Wy, by, Wy2, by2):
    s_arr = jnp.arange(E, dtype=jnp.int32)
    rem = s_arr % TILE_E
    slot_idx = (s_arr - rem) + (rem % 4) * (TILE_E // 4) + rem // 4

    def _to_slots(x):
        return jnp.take(x, slot_idx, axis=0)

    def _split(w, epw, nchm):
        m = w[:, :nchm * GCH].reshape(SC_W, nchm, GCH)
        return m, w[:, nchm * GCH:]

    src_s = _to_slots(edge_index[0])
    dst_s = _to_slots(edge_index[1])
    srcmA, srctA = _split(src_s[:HALF_A].reshape(SC_W, EPWA), EPWA, NCHA)
    srcmB, srctB = _split(src_s[HALF_A:].reshape(SC_W, EPWB), EPWB, NCHB)
    dstmA, dsttA = _split(dst_s[:HALF_A].reshape(SC_W, EPWA), EPWA, NCHA)
    dstmB, dsttB = _split(dst_s[HALF_A:].reshape(SC_W, EPWB), EPWB, NCHB)

    w0t = W0.T
    we1t = We1.T
    we2t = We2.T.astype(jnp.bfloat16)
    w1t = W1.T
    wyt = Wy.T
    wy2t = Wy2.T
    b0r = b0.reshape(1, HID)
    be1r = be1.reshape(1, E_HID)
    b2 = be2.reshape(HID, HID)
    b1r = b1.reshape(1, HID)
    bcr = bc.reshape(1, HID)
    gr = gamma.reshape(1, HID)
    betar = beta_bn.reshape(1, HID)
    byr = by.reshape(1, 2)
    by2r = by2.reshape(1, 2)
    rrep = jnp.repeat(jnp.eye(HID, dtype=jnp.bfloat16), HID, axis=1)
    stile = jnp.tile(jnp.eye(HID, dtype=jnp.bfloat16), (HID, 1))
    zer = jnp.zeros((NPT, HID), jnp.float32)

    (gA, sA), (gB, sB) = _sc_kernels()
    out = _lin0(n_feat, w0t, b0r)
    h0 = out
    for _ in range(STEPS):
        hpA = gA(out, srcmA, srctA).reshape(HALF_A // 4, 4 * HID)
        hpB = gB(out, srcmB, srctB).reshape(HALF_B // 4, 4 * HID)
        mA = _edge(e_feat, hpA, we1t, be1r, we2t, b2, rrep, stile, HALF_A, 0)
        pA = sA(mA.reshape(HALF_A, HID), dstmA, dsttA,
                zer).reshape(SC_CORES, NPAD, HID)
        mB = _edge(e_feat, hpB, we1t, be1r, we2t, b2, rrep, stile, HALF_B,
                   HALF_A // TILE_E)
        pB = sB(mB.reshape(HALF_B, HID), dstmB, dsttB,
                zer).reshape(SC_CORES, NPAD, HID)
        out = _node(pA[0, :N], pA[1, :N], pB[0, :N], pB[1, :N], out, h0,
                    w1t, b1r, bcr)
    return _bn_heads(out, gr, betar, wyt, byr, wy2t, by2r)


# speedup vs baseline: 1.0958x; 1.0020x over previous
import functools

import jax
import jax.numpy as jnp
from jax import lax
from jax.experimental import pallas as pl
from jax.experimental.pallas import tpu as pltpu
from jax.experimental.pallas import tpu_sc as plsc

N = 10000
E = 160000
D_IN = 128
HID = 32
E_IN = 16
E_HID = 128
STEPS = 2
ALPHA = 0.1
BETA = 1.0 / STEPS

SC_CORES = 2
SC_SUBCORES = 16
SC_W = SC_CORES * SC_SUBCORES
EPW = E // SC_W
GCH = 128
NCHM = 39
TAIL = EPW - NCHM * GCH
NPAD = 10240
NPT = NPAD // SC_SUBCORES

TILE_E = 3200

HALF_A = 83200
HALF_B = E - HALF_A
EPWA = HALF_A // SC_W
NCHA = EPWA // GCH
TAILA = EPWA - NCHA * GCH
EPWB = HALF_B // SC_W
NCHB = EPWB // GCH
TAILB = EPWB - NCHB * GCH


def _lin0_body(nf, w0t, b0, out):
    out[...] = jnp.maximum(
        jnp.dot(nf[...], w0t[...], preferred_element_type=jnp.float32) + b0[...],
        0.0)


def _edge_body(ef, hp, we1t, be1, we2t, b2, rrep, msgp):
    hpv = hp[...]
    h = jnp.concatenate([hpv[:, HID * q:HID * (q + 1)] for q in range(4)],
                        axis=0)
    t = jnp.maximum(
        jnp.dot(ef[...], we1t[...], preferred_element_type=jnp.float32) + be1[...],
        0.0)
    wm = jnp.dot(t.astype(jnp.bfloat16), we2t[...],
                 preferred_element_type=jnp.float32)
    hr = jnp.dot(h.astype(jnp.bfloat16), rrep[...],
                 preferred_element_type=jnp.float32)
    prod = hr * wm
    t1 = prod[:, 0:128]
    for k in range(1, 8):
        t1 = t1 + prod[:, k * 128:(k + 1) * 128]
    s = t1[:, 0:64] + t1[:, 64:128]
    m2 = jnp.dot(h, b2[...], preferred_element_type=jnp.float32)
    msg = s[:, 0:32] + s[:, 32:64] + m2
    q4 = TILE_E // 4
    msgp[...] = jnp.concatenate([msg[q4 * q:q4 * (q + 1), :] for q in range(4)],
                                axis=1)


def _node_body(a0, a1, b0_, b1_, out, h0, w1t, b1, bc, new):
    agg = (a0[...] + a1[...]) + (b0_[...] + b1_[...])
    conv = agg + out[...] + bc[...]
    temp = ALPHA * conv + (1.0 - ALPHA) * h0[...]
    lin = jnp.dot(temp, w1t[...], preferred_element_type=jnp.float32) + b1[...]
    new[...] = jnp.maximum(BETA * lin + (1.0 - BETA) * temp, 0.0)


def _bn_body(x, gamma, beta_bn, wyt, by, wy2t, by2, y1, y2):
    v = x[...]
    mu = jnp.mean(v, axis=0, keepdims=True)
    d = v - mu
    var = jnp.mean(d * d, axis=0, keepdims=True)
    yb = d * (gamma[...] * lax.rsqrt(var + 1e-5)) + beta_bn[...]
    y1[...] = jax.nn.sigmoid(
        jnp.dot(yb, wyt[...], preferred_element_type=jnp.float32) + by[...])
    y2[...] = jax.nn.sigmoid(
        jnp.dot(yb, wy2t[...], preferred_element_type=jnp.float32) + by2[...])


def _lin0(n_feat, w0t, b0):
    return pl.pallas_call(
        _lin0_body,
        out_shape=jax.ShapeDtypeStruct((N, HID), jnp.float32),
    )(n_feat, w0t, b0)


def _edge(e_feat, hp, we1t, be1, we2t, b2, rrep, half, tile_off):
    grid = (half // TILE_E,)
    fixed = lambda i: (0, 0)
    return pl.pallas_call(
        _edge_body,
        grid=grid,
        in_specs=[
            pl.BlockSpec((TILE_E, E_IN), lambda i: (i + tile_off, 0)),
            pl.BlockSpec((TILE_E // 4, 4 * HID), lambda i: (i, 0)),
            pl.BlockSpec((E_IN, E_HID), fixed),
            pl.BlockSpec((1, E_HID), fixed),
            pl.BlockSpec((E_HID, HID * HID), fixed),
            pl.BlockSpec((HID, HID), fixed),
            pl.BlockSpec((HID, HID * HID), fixed),
        ],
        out_specs=pl.BlockSpec((TILE_E // 4, 4 * HID), lambda i: (i, 0)),
        out_shape=jax.ShapeDtypeStruct((half // 4, 4 * HID), jnp.float32),
        compiler_params=pltpu.CompilerParams(
            dimension_semantics=("arbitrary",)),
    )(e_feat, hp, we1t, be1, we2t, b2, rrep)


def _node(a0, a1, b0_, b1_, out, h0, w1t, b1, bc):
    return pl.pallas_call(
        _node_body,
        out_shape=jax.ShapeDtypeStruct((N, HID), jnp.float32),
    )(a0, a1, b0_, b1_, out, h0, w1t, b1, bc)


def _bn_heads(x, gamma, beta_bn, wyt, by, wy2t, by2):
    return pl.pallas_call(
        _bn_body,
        out_shape=(jax.ShapeDtypeStruct((N, 2), jnp.float32),
                   jax.ShapeDtypeStruct((N, 2), jnp.float32)),
    )(x, gamma, beta_bn, wyt, by, wy2t, by2)


def _gather_body_fn(epw, nchm, tail):
    def body_fn(table_hbm, idxm_hbm, idxt_hbm, out_hbm, idx_v, idxt_v,
                rows_a, rows_b, rowst_v, gsem_a, gsem_b, ssem_a, ssem_b,
                tsem):
        c = lax.axis_index("c")
        s = lax.axis_index("s")
        wid = s * SC_CORES + c
        base = wid * epw
        pltpu.sync_copy(idxm_hbm.at[wid], idx_v)
        pltpu.sync_copy(idxt_hbm.at[wid], idxt_v)

        def issue(j, buf, sem):
            pltpu.async_copy(table_hbm.at[idx_v.at[j]], buf, sem)

        def wait_g(buf, sem):
            pltpu.make_async_copy(table_hbm.at[idx_v.at[0]], buf, sem).wait()

        def store(j, buf, sem):
            pltpu.async_copy(buf, out_hbm.at[pl.ds(base + j * GCH, GCH)], sem)

        def wait_s(buf, sem):
            pltpu.make_async_copy(out_hbm.at[pl.ds(base, GCH)], buf, sem).wait()

        issue(0, rows_a, gsem_a)
        issue(1, rows_b, gsem_b)

        def body(g, carry):
            j0 = 2 * g
            wait_g(rows_a, gsem_a)
            store(j0, rows_a, ssem_a)
            wait_g(rows_b, gsem_b)
            store(j0 + 1, rows_b, ssem_b)

            @pl.when(j0 + 2 < nchm)
            def _():
                wait_s(rows_a, ssem_a)
                issue(j0 + 2, rows_a, gsem_a)

            @pl.when(j0 + 3 < nchm)
            def _():
                wait_s(rows_b, ssem_b)
                issue(j0 + 3, rows_b, gsem_b)

            return carry

        lax.fori_loop(0, nchm // 2, body, 0)
        if nchm % 2:
            wait_g(rows_a, gsem_a)
            store(nchm - 1, rows_a, ssem_a)
        pltpu.async_copy(table_hbm.at[idxt_v], rowst_v, tsem).wait()
        pltpu.sync_copy(rowst_v, out_hbm.at[pl.ds(base + nchm * GCH, tail)])
        wait_s(rows_a, ssem_a)
        wait_s(rows_b, ssem_b)

    return body_fn


def _scatter_body_fn(epw, nchm, tail):
    def body_fn(msg_hbm, idxm_hbm, idxt_hbm, zer_hbm, out_hbm, idx_v,
                idxt_v, rows_a, rows_b, rowst_v, agg_sh, lsem_a, lsem_b,
                tsem):
        c = lax.axis_index("c")
        s = lax.axis_index("s")
        wid = s * SC_CORES + c
        base = wid * epw
        pltpu.sync_copy(zer_hbm, agg_sh.at[pl.ds(s * NPT, NPT)])
        pltpu.sync_copy(idxm_hbm.at[wid], idx_v)
        pltpu.sync_copy(idxt_hbm.at[wid], idxt_v)
        plsc.subcore_barrier()

        def load(j, buf, sem):
            pltpu.async_copy(msg_hbm.at[pl.ds(base + j * GCH, GCH)], buf, sem)

        def wait_l(buf, sem):
            pltpu.make_async_copy(msg_hbm.at[pl.ds(base, GCH)], buf, sem).wait()

        load(0, rows_a, lsem_a)
        load(1, rows_b, lsem_b)

        def body(g, carry):
            j0 = 2 * g
            wait_l(rows_a, lsem_a)
            pltpu.sync_copy(rows_a, agg_sh.at[idx_v.at[j0]], add=True)

            @pl.when(j0 + 2 < nchm)
            def _():
                load(j0 + 2, rows_a, lsem_a)

            wait_l(rows_b, lsem_b)
            pltpu.sync_copy(rows_b, agg_sh.at[idx_v.at[j0 + 1]], add=True)

            @pl.when(j0 + 3 < nchm)
            def _():
                load(j0 + 3, rows_b, lsem_b)

            return carry

        lax.fori_loop(0, nchm // 2, body, 0)
        if nchm % 2:
            wait_l(rows_a, lsem_a)
            pltpu.sync_copy(rows_a, agg_sh.at[idx_v.at[nchm - 1]], add=True)
        pltpu.async_copy(
            msg_hbm.at[pl.ds(base + nchm * GCH, tail)], rowst_v, tsem).wait()
        pltpu.sync_copy(rowst_v, agg_sh.at[idxt_v], add=True)
        plsc.subcore_barrier()
        pltpu.sync_copy(agg_sh.at[pl.ds(s * NPT, NPT)],
                        out_hbm.at[pl.ds(c * NPAD + s * NPT, NPT)])

    return body_fn


@functools.lru_cache(maxsize=None)
def _sc_kernels():
    mesh = plsc.VectorSubcoreMesh(core_axis_name="c", subcore_axis_name="s")
    params = pltpu.CompilerParams(use_tc_tiling_on_sc=False)

    def make_pair(half, epw, nchm, tail):
        gather = pl.kernel(
            _gather_body_fn(epw, nchm, tail),
            out_type=jax.ShapeDtypeStruct((half, HID), jnp.float32),
            mesh=mesh,
            scratch_types=[
                pltpu.VMEM((nchm, GCH), jnp.int32),
                pltpu.VMEM((tail,), jnp.int32),
                pltpu.VMEM((GCH, HID), jnp.float32),
                pltpu.VMEM((GCH, HID), jnp.float32),
                pltpu.VMEM((tail, HID), jnp.float32),
                pltpu.SemaphoreType.DMA,
                pltpu.SemaphoreType.DMA,
                pltpu.SemaphoreType.DMA,
                pltpu.SemaphoreType.DMA,
                pltpu.SemaphoreType.DMA,
            ],
            compiler_params=params,
        )
        scatter = pl.kernel(
            _scatter_body_fn(epw, nchm, tail),
            out_type=jax.ShapeDtypeStruct((SC_CORES * NPAD, HID), jnp.float32),
            mesh=mesh,
            scratch_types=[
                pltpu.VMEM((nchm, GCH), jnp.int32),
                pltpu.VMEM((tail,), jnp.int32),
                pltpu.VMEM((GCH, HID), jnp.float32),
                pltpu.VMEM((GCH, HID), jnp.float32),
                pltpu.VMEM((tail, HID), jnp.float32),
                pltpu.VMEM_SHARED((NPAD, HID), jnp.float32),
                pltpu.SemaphoreType.DMA,
                pltpu.SemaphoreType.DMA,
                pltpu.SemaphoreType.DMA,
            ],
            compiler_params=params,
        )
        return gather, scatter

    return (make_pair(HALF_A, EPWA, NCHA, TAILA),
            make_pair(HALF_B, EPWB, NCHB, TAILB))


def kernel(n_feat, e_feat, edge_index, W0, b0, We1, be1, We2, be2, bc, W1,
           b1, gamma, beta_bn, Wy, by, Wy2, by2):
    s_arr = jnp.arange(E, dtype=jnp.int32)
    rem = s_arr % TILE_E
    slot_idx = (s_arr - rem) + (rem % 4) * (TILE_E // 4) + rem // 4

    def _to_slots(x):
        return jnp.take(x, slot_idx, axis=0)

    def _split(w, epw, nchm):
        m = w[:, :nchm * GCH].reshape(SC_W, nchm, GCH)
        return m, w[:, nchm * GCH:]

    src_s = _to_slots(edge_index[0])
    dst_s = _to_slots(edge_index[1])
    srcmA, srctA = _split(src_s[:HALF_A].reshape(SC_W, EPWA), EPWA, NCHA)
    srcmB, srctB = _split(src_s[HALF_A:].reshape(SC_W, EPWB), EPWB, NCHB)
    dstmA, dsttA = _split(dst_s[:HALF_A].reshape(SC_W, EPWA), EPWA, NCHA)
    dstmB, dsttB = _split(dst_s[HALF_A:].reshape(SC_W, EPWB), EPWB, NCHB)

    w0t = W0.T
    we1t = We1.T
    we2t = We2.T.astype(jnp.bfloat16)
    w1t = W1.T
    wyt = Wy.T
    wy2t = Wy2.T
    b0r = b0.reshape(1, HID)
    be1r = be1.reshape(1, E_HID)
    b2 = be2.reshape(HID, HID)
    b1r = b1.reshape(1, HID)
    bcr = bc.reshape(1, HID)
    gr = gamma.reshape(1, HID)
    betar = beta_bn.reshape(1, HID)
    byr = by.reshape(1, 2)
    by2r = by2.reshape(1, 2)
    rrep = jnp.repeat(jnp.eye(HID, dtype=jnp.bfloat16), HID, axis=1)
    zer = jnp.zeros((NPT, HID), jnp.float32)

    (gA, sA), (gB, sB) = _sc_kernels()
    out = _lin0(n_feat, w0t, b0r)
    h0 = out
    for _ in range(STEPS):
        hpA = gA(out, srcmA, srctA).reshape(HALF_A // 4, 4 * HID)
        hpB = gB(out, srcmB, srctB).reshape(HALF_B // 4, 4 * HID)
        mA = _edge(e_feat, hpA, we1t, be1r, we2t, b2, rrep, HALF_A, 0)
        pA = sA(mA.reshape(HALF_A, HID), dstmA, dsttA,
                zer).reshape(SC_CORES, NPAD, HID)
        mB = _edge(e_feat, hpB, we1t, be1r, we2t, b2, rrep, HALF_B,
                   HALF_A // TILE_E)
        pB = sB(mB.reshape(HALF_B, HID), dstmB, dsttB,
                zer).reshape(SC_CORES, NPAD, HID)
        out = _node(pA[0, :N], pA[1, :N], pB[0, :N], pB[1, :N], out, h0,
                    w1t, b1r, bcr)
    return _bn_heads(out, gr, betar, wyt, byr, wy2t, by2r)
